# baseline (speedup 1.0000x reference)
import functools

import jax
import jax.numpy as jnp
from jax import lax
from jax.experimental import pallas as pl
from jax.experimental.pallas import tpu as pltpu
from jax.experimental.pallas import tpu_sc as plsc

N = 10000
E = 320000
D = 128
NEG_SLOPE = 0.2

NW = 32
NPW = 313
NPAD = NW * NPW
FC = 8000
NCHF = E // FC
DR = 4096
FBUF = DR + FC + 16
EROW = E + DR
FC2 = 4096
GB = 128
PADROW = NPW * D

_BLK = 1000


def _proj_body(z_ref, ws_ref, wd_ref, as_ref, ad_ref):
    z = z_ref[...]
    as_ref[...] = jnp.dot(z, ws_ref[...], preferred_element_type=jnp.float32)
    ad_ref[...] = jnp.dot(z, wd_ref[...], preferred_element_type=jnp.float32)


def _proj(z, att_s, att_d):
    return pl.pallas_call(
        _proj_body,
        grid=(N // _BLK,),
        in_specs=[
            pl.BlockSpec((_BLK, D), lambda i: (i, 0)),
            pl.BlockSpec((D, 1), lambda i: (0, 0)),
            pl.BlockSpec((D, 1), lambda i: (0, 0)),
        ],
        out_specs=[
            pl.BlockSpec((_BLK, 1), lambda i: (i, 0)),
            pl.BlockSpec((_BLK, 1), lambda i: (i, 0)),
        ],
        out_shape=[
            jax.ShapeDtypeStruct((N, 1), jnp.float32),
            jax.ShapeDtypeStruct((N, 1), jnp.float32),
        ],
    )(z, att_s, att_d)


def _combine1_body(agg_ref, z_ref, wl_ref, bl_ref, wr_ref, ws_ref, wd_ref,
                   h_ref, as_ref, ad_ref):
    acc = jnp.dot(agg_ref[...], wl_ref[...], preferred_element_type=jnp.float32)
    acc += jnp.dot(z_ref[...], wr_ref[...], preferred_element_type=jnp.float32)
    acc += bl_ref[...]
    h = jnp.maximum(acc, 0.0)
    h_ref[...] = h
    as_ref[...] = jnp.dot(h, ws_ref[...], preferred_element_type=jnp.float32)
    ad_ref[...] = jnp.dot(h, wd_ref[...], preferred_element_type=jnp.float32)


def _combine1(agg, z, wl, bl, wr, att_s, att_d):
    return pl.pallas_call(
        _combine1_body,
        grid=(N // _BLK,),
        in_specs=[
            pl.BlockSpec((_BLK, D), lambda i: (i, 0)),
            pl.BlockSpec((_BLK, D), lambda i: (i, 0)),
            pl.BlockSpec((D, D), lambda i: (0, 0)),
            pl.BlockSpec((1, D), lambda i: (0, 0)),
            pl.BlockSpec((D, D), lambda i: (0, 0)),
            pl.BlockSpec((D, 1), lambda i: (0, 0)),
            pl.BlockSpec((D, 1), lambda i: (0, 0)),
        ],
        out_specs=[
            pl.BlockSpec((_BLK, D), lambda i: (i, 0)),
            pl.BlockSpec((_BLK, 1), lambda i: (i, 0)),
            pl.BlockSpec((_BLK, 1), lambda i: (i, 0)),
        ],
        out_shape=[
            jax.ShapeDtypeStruct((N, D), jnp.float32),
            jax.ShapeDtypeStruct((N, 1), jnp.float32),
            jax.ShapeDtypeStruct((N, 1), jnp.float32),
        ],
    )(agg, z, wl, bl.reshape(1, D), wr, att_s, att_d)


def _combine2_body(agg_ref, z_ref, wl_ref, bl_ref, wr_ref, wo_ref, bo_ref, o_ref):
    acc = jnp.dot(agg_ref[...], wl_ref[...], preferred_element_type=jnp.float32)
    acc += jnp.dot(z_ref[...], wr_ref[...], preferred_element_type=jnp.float32)
    acc += bl_ref[...]
    h = jnp.maximum(acc, 0.0)
    o_ref[...] = jnp.dot(h, wo_ref[...], preferred_element_type=jnp.float32) + bo_ref[...]


def _combine2(agg, z, wl, bl, wr, wo, bo):
    return pl.pallas_call(
        _combine2_body,
        grid=(N // _BLK,),
        in_specs=[
            pl.BlockSpec((_BLK, D), lambda i: (i, 0)),
            pl.BlockSpec((_BLK, D), lambda i: (i, 0)),
            pl.BlockSpec((D, D), lambda i: (0, 0)),
            pl.BlockSpec((1, D), lambda i: (0, 0)),
            pl.BlockSpec((D, D), lambda i: (0, 0)),
            pl.BlockSpec((D, 1), lambda i: (0, 0)),
            pl.BlockSpec((1, 1), lambda i: (0, 0)),
        ],
        out_specs=pl.BlockSpec((_BLK, 1), lambda i: (i, 0)),
        out_shape=jax.ShapeDtypeStruct((N, 1), jnp.float32),
    )(agg, z, wl, bl.reshape(1, D), wr, wo, bo.reshape(1, 1))


@functools.lru_cache(maxsize=None)
def _make_filter_sc():
    mesh = plsc.VectorSubcoreMesh(core_axis_name="c", subcore_axis_name="s")
    return functools.partial(
        pl.kernel,
        mesh=mesh,
        compiler_params=pltpu.CompilerParams(needs_layout_passes=False),
        out_type=[
            jax.ShapeDtypeStruct((NW * EROW,), jnp.int32),
            jax.ShapeDtypeStruct((NW * EROW,), jnp.int32),
            jax.ShapeDtypeStruct((NW * 16,), jnp.int32),
        ],
        scratch_types=[
            pltpu.VMEM((2 * FC,), jnp.int32),
            pltpu.VMEM((2 * FC,), jnp.int32),
            pltpu.VMEM((FBUF,), jnp.int32),
            pltpu.VMEM((FBUF,), jnp.int32),
            pltpu.VMEM((16,), jnp.int32),
            pltpu.SemaphoreType.DMA,
        ],
    )(_filter_sc_body)


def _filter_sc_body(dst_hbm, src_hbm, fd_hbm, fs_hbm, cnt_hbm,
                    dvb, svb, bdst, bsrc, cbuf, semf):
    wid = lax.axis_index("s") * 2 + lax.axis_index("c")
    lo = wid * NPW
    hi = lo + NPW
    wbase = wid * EROW

    def fire(g, slot):
        pltpu.async_copy(dst_hbm.at[pl.ds(g * FC, FC)],
                         dvb.at[pl.ds(slot * FC, FC)], semf)
        pltpu.async_copy(src_hbm.at[pl.ds(g * FC, FC)],
                         svb.at[pl.ds(slot * FC, FC)], semf)

    fire(jnp.int32(0), jnp.int32(0))

    def drain(bcv, ndr):
        def do(args):
            bcv, ndr = args
            off = wbase + ndr * DR
            pltpu.sync_copy(bdst.at[pl.ds(0, DR)], fd_hbm.at[pl.ds(off, DR)])
            pltpu.sync_copy(bsrc.at[pl.ds(0, DR)], fs_hbm.at[pl.ds(off, DR)])
            nmv = (bcv[0] - DR + 15) // 16

            def mv(i, _):
                o = i * 16
                bdst[pl.ds(o, 16)] = bdst[pl.ds(DR + o, 16)]
                bsrc[pl.ds(o, 16)] = bsrc[pl.ds(DR + o, 16)]
                return _

            lax.fori_loop(0, nmv, mv, None)
            return (bcv - DR, ndr + 1)

        return lax.cond(bcv[0] >= DR, do, lambda a: a, (bcv, ndr))

    def chunk(g, carry):
        bcv, ndr = carry

        @pl.when(g + 1 < NCHF)
        def _next():
            fire(g + 1, (g + 1) % 2)

        pltpu.make_async_copy(dst_hbm.at[pl.ds(0, FC)],
                              dvb.at[pl.ds(0, FC)], semf).wait()
        pltpu.make_async_copy(src_hbm.at[pl.ds(0, FC)],
                              svb.at[pl.ds(0, FC)], semf).wait()
        sbase = (g % 2) * FC

        def filt(v, bcv):
            for u in range(4):
                o = sbase + (v * 4 + u) * 16
                d = dvb[pl.ds(o, 16)]
                m = (d >= lo) & (d < hi)
                mi = m.astype(jnp.int32)
                pos = bcv + plsc.cumsum(mi) - 1
                plsc.store_scatter(bdst, [pos], d, mask=m)
                plsc.store_scatter(bsrc, [pos], svb[pl.ds(o, 16)], mask=m)
                bcv = bcv + plsc.all_reduce_population_count(m)
            return bcv

        bcv = lax.fori_loop(0, FC // 64, filt, bcv)
        bcv, ndr = drain(bcv, ndr)
        bcv, ndr = drain(bcv, ndr)
        return (bcv, ndr)

    bcv0 = jnp.zeros((16,), jnp.int32)
    bcv, ndr = lax.fori_loop(0, NCHF, chunk, (bcv0, jnp.int32(0)))
    off = wbase + ndr * DR
    pltpu.sync_copy(bdst.at[pl.ds(0, DR)], fd_hbm.at[pl.ds(off, DR)])
    pltpu.sync_copy(bsrc.at[pl.ds(0, DR)], fs_hbm.at[pl.ds(off, DR)])
    total = ndr * DR + bcv[0]
    cbuf[pl.ds(0, 16)] = jnp.full((16,), 0, jnp.int32) + total
    pltpu.sync_copy(cbuf, cnt_hbm.at[pl.ds(wid * 16, 16)])


@functools.lru_cache(maxsize=None)
def _make_agg_sc():
    mesh = plsc.VectorSubcoreMesh(core_axis_name="c", subcore_axis_name="s")
    return functools.partial(
        pl.kernel,
        mesh=mesh,
        compiler_params=pltpu.CompilerParams(needs_layout_passes=False),
        out_type=jax.ShapeDtypeStruct((NPAD * D,), jnp.float32),
        scratch_types=[
            pltpu.VMEM((N,), jnp.float32),
            pltpu.VMEM((N,), jnp.float32),
            pltpu.VMEM((FC2,), jnp.int32),
            pltpu.VMEM((FC2,), jnp.int32),
            pltpu.VMEM((FC2,), jnp.float32),
            pltpu.VMEM((FC2,), jnp.int32),
            pltpu.VMEM((2 * GB, D), jnp.float32),
            pltpu.VMEM(((NPW + 1) * D,), jnp.float32),
            pltpu.VMEM((16,), jnp.int32),
            pltpu.SMEM((NPW + 7,), jnp.float32),
            pltpu.SemaphoreType.DMA,
        ],
    )(_agg_sc_body)


def _agg_sc_body(fd_hbm, fs_hbm, cnt_hbm, adst_hbm, asrc_hbm, z_hbm, agg_hbm,
                 adst_v, asrc_v, fdc, fsc, feb, obuf, zbuf, um, cbuf, den, sem):
    wid = lax.axis_index("s") * 2 + lax.axis_index("c")
    lo = wid * NPW
    wbase = wid * EROW

    pltpu.sync_copy(adst_hbm, adst_v)
    pltpu.sync_copy(asrc_hbm, asrc_v)
    pltpu.sync_copy(cnt_hbm.at[pl.ds(wid * 16, 16)], cbuf)
    cnt = cbuf[pl.ds(0, 16)][0]

    def init_um(i, _):
        um[pl.ds(i * 16, 16)] = jnp.full((16,), -3e38, jnp.float32)
        return _
    lax.fori_loop(0, (NPW + 1) * D // 16, init_um, None)

    def init_den(i, _):
        den[i] = 0.0
        return _
    lax.fori_loop(0, NPW, init_den, None)

    def chunk(c, _):
        pltpu.sync_copy(fd_hbm.at[pl.ds(wbase + c * FC2, FC2)], fdc)
        pltpu.sync_copy(fs_hbm.at[pl.ds(wbase + c * FC2, FC2)], fsc)
        ne = jnp.minimum(FC2, cnt - c * FC2)
        nv = (ne + 15) // 16
        nb = (ne + GB - 1) // GB

        def escore(v, _):
            o = v * 16
            d = fdc[pl.ds(o, 16)]
            s = fsc[pl.ds(o, 16)]
            valid = (lax.iota(jnp.int32, 16) + o) < ne
            safe = lax.iota(jnp.int32, 16) + (v & 511) * 16
            s = jnp.where(valid, s, safe)
            fsc[pl.ds(o, 16)] = s
            a = plsc.load_gather(adst_v, [jnp.where(valid, d, 0)]) + \
                plsc.load_gather(asrc_v, [s])
            a = jnp.where(a > 0.0, a, NEG_SLOPE * a)
            a = jnp.minimum(a, 60.0)
            feb[pl.ds(o, 16)] = jnp.where(valid, jnp.exp(a), 0.0)
            obuf[pl.ds(o, 16)] = jnp.where(valid, (d - lo) * D, PADROW)
            return _

        lax.fori_loop(0, nv, escore, None)

        def sanitize(v, _):
            o = v * 16
            fsc[pl.ds(o, 16)] = lax.iota(jnp.int32, 16) + (v & 511) * 16
            return _

        lax.fori_loop(nv, nb * (GB // 16), sanitize, None)

        def fire(j, slot):
            pltpu.async_copy(
                z_hbm.at[fsc.at[pl.ds(j * GB, GB)]],
                zbuf.at[pl.ds(slot * GB, GB)], sem)

        @pl.when(nb > 0)
        def _prologue():
            fire(jnp.int32(0), jnp.int32(0))

        def batch(j, _):
            @pl.when(j + 1 < nb)
            def _next():
                fire(j + 1, (j + 1) % 2)

            pltpu.make_async_copy(
                z_hbm.at[fsc.at[pl.ds(0, GB)]],
                zbuf.at[pl.ds(0, GB)], sem).wait()

            slotbase = (j % 2) * GB - j * GB
            kend = jnp.minimum(ne, (j + 1) * GB)
            ng = (kend - j * GB + 15) // 16

            def group(g, _):
                o = j * GB + g * 16
                ov = obuf[pl.ds(o, 16)]
                ev = feb[pl.ds(o, 16)]
                for jj in range(16):
                    ub = ov[jj]
                    eb = jnp.full((16,), ev[jj], jnp.float32)
                    ri = slotbase + o + jj
                    dl = jnp.right_shift(ub, 7)
                    zvs = [zbuf[ri, pl.ds(dd * 16, 16)] for dd in range(D // 16)]
                    accs = [um[pl.ds(ub + dd * 16, 16)] for dd in range(D // 16)]
                    for dd in range(D // 16):
                        um[pl.ds(ub + dd * 16, 16)] = jnp.maximum(
                            accs[dd], zvs[dd] * eb)
                    den[dl] = den[dl] + ev[jj]
                return _

            lax.fori_loop(0, ng, group, None)
            return _

        lax.fori_loop(0, nb, batch, None)
        return _

    nc = (cnt + FC2 - 1) // FC2
    lax.fori_loop(0, nc, chunk, None)

    def norm(i, _):
        dd = den[i]
        db = jnp.full((16,), dd, jnp.float32)
        rb = jnp.where(db == 0.0, jnp.zeros((16,), jnp.float32),
                       jnp.ones((16,), jnp.float32) / (db + 1e-16))
        for jj in range(D // 16):
            uo = i * D + jj * 16
            um[pl.ds(uo, 16)] = um[pl.ds(uo, 16)] * rb
        return _

    lax.fori_loop(0, NPW, norm, None)
    pltpu.sync_copy(um.at[pl.ds(0, NPW * D)], agg_hbm.at[pl.ds(lo * D, NPW * D)])


def kernel(x, edge_index, lin_l0_w, lin_l0_b, lin_r0_w, att_src0, att_dst0,
           lin_l1_w, lin_l1_b, lin_r1_w, att_src1, att_dst1, out_w, out_b):
    src = edge_index[0]
    dst = edge_index[1]

    filter_sc = _make_filter_sc()
    agg_sc = _make_agg_sc()

    fd, fs, cnts = filter_sc(dst, src)
    as0, ad0 = _proj(x, att_src0, att_dst0)
    agg0 = agg_sc(fd, fs, cnts, as0.reshape(N), ad0.reshape(N), x)
    agg0 = agg0.reshape(NPAD, D)[:N]
    h, as1, ad1 = _combine1(agg0, x, lin_l0_w, lin_l0_b, lin_r0_w,
                            att_src1, att_dst1)
    agg1 = agg_sc(fd, fs, cnts, as1.reshape(N), ad1.reshape(N), h)
    agg1 = agg1.reshape(NPAD, D)[:N]
    return _combine2(agg1, h, lin_l1_w, lin_l1_b, lin_r1_w, out_w, out_b)


# speedup vs baseline: 1.2262x; 1.2262x over previous
import functools

import jax
import jax.numpy as jnp
from jax import lax
from jax.experimental import pallas as pl
from jax.experimental.pallas import tpu as pltpu
from jax.experimental.pallas import tpu_sc as plsc

N = 10000
E = 320000
D = 128
NEG_SLOPE = 0.2

NW = 32
NPW = 313
NPAD = NW * NPW
FC = 8000
NCHF = E // FC
DR = 4096
FBUF = DR + FC // 2 + 16
EROW = E // 2 + DR
FC2 = 4096
GB = 128
PADROW = NPW * D

_BLK = 1000


def _proj_body(z_ref, ws_ref, wd_ref, as_ref, ad_ref):
    z = z_ref[...]
    as_ref[...] = jnp.dot(z, ws_ref[...], preferred_element_type=jnp.float32)
    ad_ref[...] = jnp.dot(z, wd_ref[...], preferred_element_type=jnp.float32)


def _proj(z, att_s, att_d):
    return pl.pallas_call(
        _proj_body,
        grid=(N // _BLK,),
        in_specs=[
            pl.BlockSpec((_BLK, D), lambda i: (i, 0)),
            pl.BlockSpec((D, 1), lambda i: (0, 0)),
            pl.BlockSpec((D, 1), lambda i: (0, 0)),
        ],
        out_specs=[
            pl.BlockSpec((_BLK, 1), lambda i: (i, 0)),
            pl.BlockSpec((_BLK, 1), lambda i: (i, 0)),
        ],
        out_shape=[
            jax.ShapeDtypeStruct((N, 1), jnp.float32),
            jax.ShapeDtypeStruct((N, 1), jnp.float32),
        ],
    )(z, att_s, att_d)


def _combine1_body(agg_ref, z_ref, wl_ref, bl_ref, wr_ref, ws_ref, wd_ref,
                   h_ref, as_ref, ad_ref):
    acc = jnp.dot(agg_ref[...], wl_ref[...], preferred_element_type=jnp.float32)
    acc += jnp.dot(z_ref[...], wr_ref[...], preferred_element_type=jnp.float32)
    acc += bl_ref[...]
    h = jnp.maximum(acc, 0.0)
    h_ref[...] = h
    as_ref[...] = jnp.dot(h, ws_ref[...], preferred_element_type=jnp.float32)
    ad_ref[...] = jnp.dot(h, wd_ref[...], preferred_element_type=jnp.float32)


def _combine1(agg, z, wl, bl, wr, att_s, att_d):
    return pl.pallas_call(
        _combine1_body,
        grid=(N // _BLK,),
        in_specs=[
            pl.BlockSpec((_BLK, D), lambda i: (i, 0)),
            pl.BlockSpec((_BLK, D), lambda i: (i, 0)),
            pl.BlockSpec((D, D), lambda i: (0, 0)),
            pl.BlockSpec((1, D), lambda i: (0, 0)),
            pl.BlockSpec((D, D), lambda i: (0, 0)),
            pl.BlockSpec((D, 1), lambda i: (0, 0)),
            pl.BlockSpec((D, 1), lambda i: (0, 0)),
        ],
        out_specs=[
            pl.BlockSpec((_BLK, D), lambda i: (i, 0)),
            pl.BlockSpec((_BLK, 1), lambda i: (i, 0)),
            pl.BlockSpec((_BLK, 1), lambda i: (i, 0)),
        ],
        out_shape=[
            jax.ShapeDtypeStruct((N, D), jnp.float32),
            jax.ShapeDtypeStruct((N, 1), jnp.float32),
            jax.ShapeDtypeStruct((N, 1), jnp.float32),
        ],
    )(agg, z, wl, bl.reshape(1, D), wr, att_s, att_d)


def _combine2_body(agg_ref, z_ref, wl_ref, bl_ref, wr_ref, wo_ref, bo_ref, o_ref):
    acc = jnp.dot(agg_ref[...], wl_ref[...], preferred_element_type=jnp.float32)
    acc += jnp.dot(z_ref[...], wr_ref[...], preferred_element_type=jnp.float32)
    acc += bl_ref[...]
    h = jnp.maximum(acc, 0.0)
    o_ref[...] = jnp.dot(h, wo_ref[...], preferred_element_type=jnp.float32) + bo_ref[...]


def _combine2(agg, z, wl, bl, wr, wo, bo):
    return pl.pallas_call(
        _combine2_body,
        grid=(N // _BLK,),
        in_specs=[
            pl.BlockSpec((_BLK, D), lambda i: (i, 0)),
            pl.BlockSpec((_BLK, D), lambda i: (i, 0)),
            pl.BlockSpec((D, D), lambda i: (0, 0)),
            pl.BlockSpec((1, D), lambda i: (0, 0)),
            pl.BlockSpec((D, D), lambda i: (0, 0)),
            pl.BlockSpec((D, 1), lambda i: (0, 0)),
            pl.BlockSpec((1, 1), lambda i: (0, 0)),
        ],
        out_specs=pl.BlockSpec((_BLK, 1), lambda i: (i, 0)),
        out_shape=jax.ShapeDtypeStruct((N, 1), jnp.float32),
    )(agg, z, wl, bl.reshape(1, D), wr, wo, bo.reshape(1, 1))


@functools.lru_cache(maxsize=None)
def _make_filter_sc():
    mesh = plsc.VectorSubcoreMesh(core_axis_name="c", subcore_axis_name="s")
    return functools.partial(
        pl.kernel,
        mesh=mesh,
        compiler_params=pltpu.CompilerParams(needs_layout_passes=False),
        out_type=[
            jax.ShapeDtypeStruct((NW * 2 * EROW,), jnp.int32),
            jax.ShapeDtypeStruct((NW * 2 * EROW,), jnp.int32),
            jax.ShapeDtypeStruct((NW * 32,), jnp.int32),
        ],
        scratch_types=[
            pltpu.VMEM((2 * FC,), jnp.int32),
            pltpu.VMEM((2 * FC,), jnp.int32),
            pltpu.VMEM((FBUF,), jnp.int32),
            pltpu.VMEM((FBUF,), jnp.int32),
            pltpu.VMEM((FBUF,), jnp.int32),
            pltpu.VMEM((FBUF,), jnp.int32),
            pltpu.VMEM((16,), jnp.int32),
            pltpu.SemaphoreType.DMA,
        ],
    )(_filter_sc_body)


def _filter_sc_body(dst_hbm, src_hbm, fd_hbm, fs_hbm, cnt_hbm,
                    dvb, svb, bdstA, bsrcA, bdstB, bsrcB, cbuf, semf):
    wid = lax.axis_index("s") * 2 + lax.axis_index("c")
    lo = wid * NPW
    hi = lo + NPW
    wbaseA = wid * 2 * EROW
    wbaseB = wbaseA + EROW

    def fire(g, slot):
        pltpu.async_copy(dst_hbm.at[pl.ds(g * FC, FC)],
                         dvb.at[pl.ds(slot * FC, FC)], semf)
        pltpu.async_copy(src_hbm.at[pl.ds(g * FC, FC)],
                         svb.at[pl.ds(slot * FC, FC)], semf)

    fire(jnp.int32(0), jnp.int32(0))

    def mk_drain(bdst, bsrc, wbase):
        def drain(bc, ndr):
            def do(args):
                bc, ndr = args
                off = wbase + ndr * DR
                pltpu.sync_copy(bdst.at[pl.ds(0, DR)], fd_hbm.at[pl.ds(off, DR)])
                pltpu.sync_copy(bsrc.at[pl.ds(0, DR)], fs_hbm.at[pl.ds(off, DR)])
                nmv = (bc - DR + 15) // 16

                def mv(i, _):
                    o = i * 16
                    bdst[pl.ds(o, 16)] = bdst[pl.ds(DR + o, 16)]
                    bsrc[pl.ds(o, 16)] = bsrc[pl.ds(DR + o, 16)]
                    return _

                lax.fori_loop(0, nmv, mv, None)
                return (bc - DR, ndr + 1)

            return lax.cond(bc >= DR, do, lambda a: a, (bc, ndr))
        return drain

    drainA = mk_drain(bdstA, bsrcA, wbaseA)
    drainB = mk_drain(bdstB, bsrcB, wbaseB)

    def chunk(g, carry):
        bcA, ndrA, bcB, ndrB = carry

        @pl.when(g + 1 < NCHF)
        def _next():
            fire(g + 1, (g + 1) % 2)

        pltpu.make_async_copy(dst_hbm.at[pl.ds(0, FC)],
                              dvb.at[pl.ds(0, FC)], semf).wait()
        pltpu.make_async_copy(src_hbm.at[pl.ds(0, FC)],
                              svb.at[pl.ds(0, FC)], semf).wait()
        sbase = (g % 2) * FC

        def filt(v, carry):
            bcA, bcB = carry
            o = sbase + v * 32
            d0 = dvb[pl.ds(o, 16)]
            d1 = dvb[pl.ds(o + 16, 16)]
            m0 = (d0 >= lo) & (d0 < hi)
            m1 = (d1 >= lo) & (d1 < hi)
            plsc.store_compressed(bdstA.at[pl.ds(bcA, 16)], d0, mask=m0)
            plsc.store_compressed(bsrcA.at[pl.ds(bcA, 16)],
                                  svb[pl.ds(o, 16)], mask=m0)
            plsc.store_compressed(bdstB.at[pl.ds(bcB, 16)], d1, mask=m1)
            plsc.store_compressed(bsrcB.at[pl.ds(bcB, 16)],
                                  svb[pl.ds(o + 16, 16)], mask=m1)
            pc0 = plsc.all_reduce_population_count(m0)
            pc1 = plsc.all_reduce_population_count(m1)
            return (bcA + pc0[0], bcB + pc1[0])

        bcA, bcB = lax.fori_loop(0, FC // 32, filt, (bcA, bcB))
        bcA, ndrA = drainA(bcA, ndrA)
        bcB, ndrB = drainB(bcB, ndrB)
        return (bcA, ndrA, bcB, ndrB)

    z0 = jnp.int32(0)
    bcA, ndrA, bcB, ndrB = lax.fori_loop(0, NCHF, chunk, (z0, z0, z0, z0))
    offA = wbaseA + ndrA * DR
    pltpu.sync_copy(bdstA.at[pl.ds(0, DR)], fd_hbm.at[pl.ds(offA, DR)])
    pltpu.sync_copy(bsrcA.at[pl.ds(0, DR)], fs_hbm.at[pl.ds(offA, DR)])
    offB = wbaseB + ndrB * DR
    pltpu.sync_copy(bdstB.at[pl.ds(0, DR)], fd_hbm.at[pl.ds(offB, DR)])
    pltpu.sync_copy(bsrcB.at[pl.ds(0, DR)], fs_hbm.at[pl.ds(offB, DR)])
    cbuf[pl.ds(0, 16)] = jnp.full((16,), 0, jnp.int32) + (ndrA * DR + bcA)
    pltpu.sync_copy(cbuf, cnt_hbm.at[pl.ds(wid * 32, 16)])
    cbuf[pl.ds(0, 16)] = jnp.full((16,), 0, jnp.int32) + (ndrB * DR + bcB)
    pltpu.sync_copy(cbuf, cnt_hbm.at[pl.ds(wid * 32 + 16, 16)])


@functools.lru_cache(maxsize=None)
def _make_agg_sc():
    mesh = plsc.VectorSubcoreMesh(core_axis_name="c", subcore_axis_name="s")
    return functools.partial(
        pl.kernel,
        mesh=mesh,
        compiler_params=pltpu.CompilerParams(needs_layout_passes=False),
        out_type=jax.ShapeDtypeStruct((NPAD * D,), jnp.float32),
        scratch_types=[
            pltpu.VMEM((N,), jnp.float32),
            pltpu.VMEM((N,), jnp.float32),
            pltpu.VMEM((FC2,), jnp.int32),
            pltpu.VMEM((FC2,), jnp.int32),
            pltpu.VMEM((FC2,), jnp.float32),
            pltpu.VMEM((FC2,), jnp.int32),
            pltpu.VMEM((2 * GB, D), jnp.float32),
            pltpu.VMEM(((NPW + 1) * D,), jnp.float32),
            pltpu.VMEM((16,), jnp.int32),
            pltpu.SMEM((NPW + 7,), jnp.float32),
            pltpu.SemaphoreType.DMA,
        ],
    )(_agg_sc_body)


def _agg_sc_body(fd_hbm, fs_hbm, cnt_hbm, adst_hbm, asrc_hbm, z_hbm, agg_hbm,
                 adst_v, asrc_v, fdc, fsc, feb, obuf, zbuf, um, cbuf, den, sem):
    wid = lax.axis_index("s") * 2 + lax.axis_index("c")
    lo = wid * NPW

    pltpu.sync_copy(adst_hbm, adst_v)
    pltpu.sync_copy(asrc_hbm, asrc_v)
    pltpu.sync_copy(cnt_hbm.at[pl.ds(wid * 32, 16)], cbuf)
    cntA = cbuf[pl.ds(0, 16)][0]
    pltpu.sync_copy(cnt_hbm.at[pl.ds(wid * 32 + 16, 16)], cbuf)
    cntB = cbuf[pl.ds(0, 16)][0]

    def init_um(i, _):
        um[pl.ds(i * 16, 16)] = jnp.full((16,), -3e38, jnp.float32)
        return _
    lax.fori_loop(0, (NPW + 1) * D // 16, init_um, None)

    def init_den(i, _):
        den[i] = 0.0
        return _
    lax.fori_loop(0, NPW, init_den, None)

    def process_list(wbase, cnt):
      def chunk(c, _):
        pltpu.sync_copy(fd_hbm.at[pl.ds(wbase + c * FC2, FC2)], fdc)
        pltpu.sync_copy(fs_hbm.at[pl.ds(wbase + c * FC2, FC2)], fsc)
        ne = jnp.minimum(FC2, cnt - c * FC2)
        nv = (ne + 15) // 16
        nb = (ne + GB - 1) // GB

        def escore(v, _):
            o = v * 16
            d = fdc[pl.ds(o, 16)]
            s = fsc[pl.ds(o, 16)]
            valid = (lax.iota(jnp.int32, 16) + o) < ne
            safe = lax.iota(jnp.int32, 16) + (v & 511) * 16
            s = jnp.where(valid, s, safe)
            fsc[pl.ds(o, 16)] = s
            a = plsc.load_gather(adst_v, [jnp.where(valid, d, 0)]) + \
                plsc.load_gather(asrc_v, [s])
            a = jnp.where(a > 0.0, a, NEG_SLOPE * a)
            a = jnp.minimum(a, 60.0)
            feb[pl.ds(o, 16)] = jnp.where(valid, jnp.exp(a), 0.0)
            obuf[pl.ds(o, 16)] = jnp.where(valid, (d - lo) * D, PADROW)
            return _

        lax.fori_loop(0, nv, escore, None)

        def sanitize(v, _):
            o = v * 16
            fsc[pl.ds(o, 16)] = lax.iota(jnp.int32, 16) + (v & 511) * 16
            return _

        lax.fori_loop(nv, nb * (GB // 16), sanitize, None)

        def fire(j, slot):
            pltpu.async_copy(
                z_hbm.at[fsc.at[pl.ds(j * GB, GB)]],
                zbuf.at[pl.ds(slot * GB, GB)], sem)

        @pl.when(nb > 0)
        def _prologue():
            fire(jnp.int32(0), jnp.int32(0))

        def batch(j, _):
            @pl.when(j + 1 < nb)
            def _next():
                fire(j + 1, (j + 1) % 2)

            pltpu.make_async_copy(
                z_hbm.at[fsc.at[pl.ds(0, GB)]],
                zbuf.at[pl.ds(0, GB)], sem).wait()

            slotbase = (j % 2) * GB - j * GB
            kend = jnp.minimum(ne, (j + 1) * GB)
            ng = (kend - j * GB + 15) // 16

            def group(g, _):
                o = j * GB + g * 16
                ov = obuf[pl.ds(o, 16)]
                ev = feb[pl.ds(o, 16)]
                for jj in range(16):
                    ub = ov[jj]
                    eb = jnp.full((16,), ev[jj], jnp.float32)
                    ri = slotbase + o + jj
                    dl = jnp.right_shift(ub, 7)
                    zvs = [zbuf[ri, pl.ds(dd * 16, 16)] for dd in range(D // 16)]
                    accs = [um[pl.ds(ub + dd * 16, 16)] for dd in range(D // 16)]
                    for dd in range(D // 16):
                        um[pl.ds(ub + dd * 16, 16)] = jnp.maximum(
                            accs[dd], zvs[dd] * eb)
                    den[dl] = den[dl] + ev[jj]
                return _

            lax.fori_loop(0, ng, group, None)
            return _

        lax.fori_loop(0, nb, batch, None)
        return _

      nc = (cnt + FC2 - 1) // FC2
      lax.fori_loop(0, nc, chunk, None)

    process_list(wid * 2 * EROW, cntA)
    process_list(wid * 2 * EROW + EROW, cntB)

    def norm(i, _):
        dd = den[i]
        db = jnp.full((16,), dd, jnp.float32)
        rb = jnp.where(db == 0.0, jnp.zeros((16,), jnp.float32),
                       jnp.ones((16,), jnp.float32) / (db + 1e-16))
        for jj in range(D // 16):
            uo = i * D + jj * 16
            um[pl.ds(uo, 16)] = um[pl.ds(uo, 16)] * rb
        return _

    lax.fori_loop(0, NPW, norm, None)
    pltpu.sync_copy(um.at[pl.ds(0, NPW * D)], agg_hbm.at[pl.ds(lo * D, NPW * D)])


def kernel(x, edge_index, lin_l0_w, lin_l0_b, lin_r0_w, att_src0, att_dst0,
           lin_l1_w, lin_l1_b, lin_r1_w, att_src1, att_dst1, out_w, out_b):
    src = edge_index[0]
    dst = edge_index[1]

    filter_sc = _make_filter_sc()
    agg_sc = _make_agg_sc()

    fd, fs, cnts = filter_sc(dst, src)
    as0, ad0 = _proj(x, att_src0, att_dst0)
    agg0 = agg_sc(fd, fs, cnts, as0.reshape(N), ad0.reshape(N), x)
    agg0 = agg0.reshape(NPAD, D)[:N]
    h, as1, ad1 = _combine1(agg0, x, lin_l0_w, lin_l0_b, lin_r0_w,
                            att_src1, att_dst1)
    agg1 = agg_sc(fd, fs, cnts, as1.reshape(N), ad1.reshape(N), h)
    agg1 = agg1.reshape(NPAD, D)[:N]
    return _combine2(agg1, h, lin_l1_w, lin_l1_b, lin_r1_w, out_w, out_b)
